# baseline (speedup 1.0000x reference)
import jax
import jax.numpy as jnp
from jax.experimental import pallas as pl
from jax.experimental.pallas import tpu as pltpu


def _vec_index(num_qk_layer, num_v_layer):
    names = ["qkx_w1n", "qkx_b1", "qkx_b2", "vx_w1h", "vx_b1", "vx_b2"]
    for l in range(num_qk_layer):
        names += [f"qkl_alpha_{l}", f"qkl_b1_{l}", f"qkl_b2_{l}",
                  f"qkl_g_{l}", f"qkl_be_{l}"]
    for l in range(num_v_layer):
        names += [f"vl_beta_{l}", f"vl_b1_{l}", f"vl_b2_{l}",
                  f"vl_g_{l}", f"vl_be_{l}"]
    names += ["ffn_b1", "ffn_b2", "an_g", "an_b", "n_g", "n_b"]
    return {n: i for i, n in enumerate(names)}


def kernel(h_index, r_index, x, z, a_rows, noise, qkz_W, qkz_b, qkx_W1a,
           qkx_W2, vx_W1a, vx_W2, qkl_W1, qkl_W2, vl_zW, vl_zb, vl_W1,
           vl_W2, toqk_W, toqk_b, ffn_W1, ffn_W2, mbd, mbd2, vecs):
    B, V, D = x.shape
    R = qkz_W.shape[1] // D
    H = 4
    dh = D // H
    Lqk = qkl_W1.shape[0]
    Lv = vl_W1.shape[0]
    NB = 4 if B % 4 == 0 else 1
    Vf = float(V)
    vid = _vec_index(Lqk, Lv)
    bf16 = jnp.bfloat16

    a_cols = jnp.transpose(a_rows.reshape(R, V, V), (1, 0, 2)).reshape(V, R * V)
    a_cols = a_cols.astype(bf16)
    hid2 = jnp.arange(2 * D, dtype=jnp.int32) // dh
    pool = (hid2[:, None] == jnp.arange(2 * H, dtype=jnp.int32)[None, :])
    pool = pool.astype(bf16)
    poolT = jnp.transpose(pool)
    bid = jnp.arange(NB * V, dtype=jnp.int32) // V
    sel = (jnp.arange(NB, dtype=jnp.int32)[:, None] == bid[None, :])
    sel = sel.astype(bf16)
    selT = jnp.transpose(sel)

    z3 = z.reshape(B, 1, D)

    consts = dict(
        a_cols=a_cols, vecs=vecs, mbd=mbd,
        pool=pool, poolT=poolT, sel=sel, selT=selT,
        qkz_W=qkz_W, qkz_b=qkz_b,
        qkx_W1a=qkx_W1a.astype(bf16), qkx_W2=qkx_W2.astype(bf16),
        vx_W1a=vx_W1a.astype(bf16), vx_W2=vx_W2.astype(bf16),
        qkl_W1=qkl_W1.astype(bf16), qkl_W2=qkl_W2.astype(bf16),
        vl_zW=vl_zW, vl_zb=vl_zb,
        vl_W1=vl_W1.astype(bf16), vl_W2=vl_W2.astype(bf16),
        toqk_W=toqk_W.astype(bf16), toqk_b=toqk_b,
        ffn_W1=ffn_W1.astype(bf16), ffn_W2=ffn_W2.astype(bf16),
    )
    batched = dict(x=x, z=z3, noise=noise)
    names_b = list(batched.keys())
    names_c = list(consts.keys())
    arrays = {**batched, **consts}
    names = names_b + names_c

    def make_spec(n):
        a = arrays[n]
        nd = a.ndim
        if n in batched:
            return pl.BlockSpec((NB,) + a.shape[1:],
                                lambda b, *_: (b,) + (0,) * (nd - 1))
        return pl.BlockSpec(a.shape, lambda b, *_: (0,) * nd,
                            pipeline_mode=pl.Buffered(1))

    def _ln(t, g, be):
        mu = jnp.mean(t, axis=-1, keepdims=True)
        tc = t - mu
        var = jnp.mean(tc * tc, axis=-1, keepdims=True)
        return tc * jax.lax.rsqrt(var + 1e-5) * g + be

    def _body(h_ref, r_ref, *refs):
        rd = dict(zip(names, refs[:len(names)]))
        o_ref = refs[len(names)]
        base = pl.program_id(0) * NB

        xv = rd["x"][...].reshape(NB * V, D)
        z4 = rd["z"][...].reshape(NB, D)
        nz = rd["noise"][...].reshape(NB * V, 1)
        acols = rd["a_cols"][...]
        vecs_v = rd["vecs"][...]
        mbd_v = rd["mbd"][...]
        pool_v = rd["pool"][...]
        poolT_v = rd["poolT"][...]
        sel_v = rd["sel"][...]
        selT_v = rd["selT"][...]

        rows = jax.lax.broadcasted_iota(jnp.int32, (NB * V, 1), 0)
        onehot = jnp.zeros((NB * V, 1), jnp.float32)
        kscale_col = jnp.zeros((NB * V, 1), jnp.float32)
        for k in range(NB):
            h_k = h_ref[base + k]
            r_k = r_ref[base + k]
            s_k = jnp.where(jnp.logical_or(r_k == 2, r_k == 3),
                            jnp.float32(0.333), jnp.float32(1.0))
            in_k = jnp.logical_and(rows >= k * V, rows < (k + 1) * V)
            onehot = onehot + (rows == h_k + k * V).astype(jnp.float32)
            kscale_col = kscale_col + s_k * in_k.astype(jnp.float32)

        def vrow(name):
            i = vid[name]
            return vecs_v[i:i + 1, :]

        def bmm(a, w):
            return jnp.dot(a.astype(bf16), w, preferred_element_type=jnp.float32)

        def mlp2(t, w1, b1, w2, b2):
            h = jnp.maximum(bmm(t, w1) + b1, 0.0)
            return bmm(h, w2) + b2

        def rspmm(xcur, zrows):
            outs = []
            for k in range(NB):
                xk = xcur[k * V:(k + 1) * V]
                zk = zrows[k:k + 1]
                xz = jnp.concatenate(
                    [xk * zk[:, r * D:(r + 1) * D] for r in range(R)], axis=0)
                outs.append(jnp.dot(acols, xz.astype(bf16),
                                    preferred_element_type=jnp.float32))
            return jnp.concatenate(outs, axis=0)

        qk_zrows = (jnp.dot(z4, rd["qkz_W"][...],
                            preferred_element_type=jnp.float32)
                    + rd["qkz_b"][...])

        h1 = bmm(xv, rd["qkx_W1a"][...]) + nz * vrow("qkx_w1n") + vrow("qkx_b1")
        h1 = jnp.maximum(h1, 0.0)
        qk_x = bmm(h1, rd["qkx_W2"][...]) + vrow("qkx_b2")

        for l in range(Lqk):
            out = rspmm(qk_x, qk_zrows)
            shortcut = qk_x
            t = out + vrow(f"qkl_alpha_{l}") * qk_x
            t = mlp2(t, rd["qkl_W1"][l], vrow(f"qkl_b1_{l}"),
                     rd["qkl_W2"][l], vrow(f"qkl_b2_{l}"))
            t = _ln(t, vrow(f"qkl_g_{l}"), vrow(f"qkl_be_{l}"))
            qk_x = t + shortcut

        h1 = bmm(xv, rd["vx_W1a"][...]) + onehot * vrow("vx_w1h") + vrow("vx_b1")
        h1 = jnp.maximum(h1, 0.0)
        v_x = bmm(h1, rd["vx_W2"][...]) + vrow("vx_b2")

        for l in range(Lv):
            v_zrows = (jnp.dot(z4, rd["vl_zW"][l],
                               preferred_element_type=jnp.float32)
                       + rd["vl_zb"][l])
            out = rspmm(v_x, v_zrows)
            shortcut = v_x
            t = out + vrow(f"vl_beta_{l}") * v_x
            t = mlp2(t, rd["vl_W1"][l], vrow(f"vl_b1_{l}"),
                     rd["vl_W2"][l], vrow(f"vl_b2_{l}"))
            t = _ln(t, vrow(f"vl_g_{l}"), vrow(f"vl_be_{l}"))
            v_x = t + shortcut

        qk_cat = bmm(qk_x, rd["toqk_W"][...]) + rd["toqk_b"][...]
        s2 = qk_cat * qk_cat
        pooled = bmm(s2, pool_v)
        sq = bmm(pooled, poolT_v)
        qk_n = qk_cat * jax.lax.rsqrt(jnp.maximum(sq, 1e-24))
        qn = qk_n[:, :D]
        kw = qk_n[:, D:] * kscale_col
        poolD = pool_v[:D, :H]
        poolDT = poolT_v[:H, :D]

        attns = []
        for k in range(NB):
            sl = slice(k * V, (k + 1) * V)
            qn_k, kw_k, v_k = qn[sl], kw[sl], v_x[sl]
            kvs = jax.lax.dot_general(kw_k.astype(bf16), v_k.astype(bf16),
                                      (((0,), (0,)), ((), ())),
                                      preferred_element_type=jnp.float32)
            num_mat = bmm(qn_k, (kvs * mbd_v))
            ksum = jnp.sum(kw_k, axis=0, keepdims=True)
            den_p = bmm(qn_k * ksum, poolD)
            den = bmm(den_p, poolDT) + 2.0 * Vf
            vsum = jnp.sum(v_k, axis=0, keepdims=True)
            num = num_mat + vsum + v_k * Vf
            attns.append(num * pl.reciprocal(den, approx=True))
        attn_out = jnp.concatenate(attns, axis=0)

        xa = xv + attn_out
        xa = _ln(xa, vrow("an_g"), vrow("an_b"))
        ffn_out = mlp2(xa, rd["ffn_W1"][...], vrow("ffn_b1"),
                       rd["ffn_W2"][...], vrow("ffn_b2"))
        xo = _ln(xa + ffn_out, vrow("n_g"), vrow("n_b"))
        o_ref[...] = xo.reshape(NB, V, D)

    const_bytes = sum(int(arrays[n].size) * arrays[n].dtype.itemsize
                      for n in names_c)
    step_bytes = (sum(int(arrays[n].size) * arrays[n].dtype.itemsize
                      * NB // B for n in names_b) + NB * V * D * 4)
    vmem_limit = int(min(100 << 20, const_bytes + 4 * step_bytes + (24 << 20)))

    grid_spec = pltpu.PrefetchScalarGridSpec(
        num_scalar_prefetch=2,
        grid=(B // NB,),
        in_specs=[make_spec(n) for n in names],
        out_specs=pl.BlockSpec((NB, V, D), lambda b, *_: (b, 0, 0)),
    )
    return pl.pallas_call(
        _body,
        out_shape=jax.ShapeDtypeStruct((B, V, D), jnp.float32),
        grid_spec=grid_spec,
        compiler_params=pltpu.CompilerParams(
            dimension_semantics=("parallel",),
            vmem_limit_bytes=vmem_limit),
    )(h_index.astype(jnp.int32), r_index.astype(jnp.int32),
      *[arrays[n] for n in names])


# speedup vs baseline: 1.0282x; 1.0282x over previous
import jax
import jax.numpy as jnp
from jax.experimental import pallas as pl
from jax.experimental.pallas import tpu as pltpu


def _vec_index(num_qk_layer, num_v_layer):
    names = ["qkx_w1n", "qkx_b1", "qkx_b2", "vx_w1h", "vx_b1", "vx_b2"]
    for l in range(num_qk_layer):
        names += [f"qkl_alpha_{l}", f"qkl_b1_{l}", f"qkl_b2_{l}",
                  f"qkl_g_{l}", f"qkl_be_{l}"]
    for l in range(num_v_layer):
        names += [f"vl_beta_{l}", f"vl_b1_{l}", f"vl_b2_{l}",
                  f"vl_g_{l}", f"vl_be_{l}"]
    names += ["ffn_b1", "ffn_b2", "an_g", "an_b", "n_g", "n_b"]
    return {n: i for i, n in enumerate(names)}


def kernel(h_index, r_index, x, z, a_rows, noise, qkz_W, qkz_b, qkx_W1a,
           qkx_W2, vx_W1a, vx_W2, qkl_W1, qkl_W2, vl_zW, vl_zb, vl_W1,
           vl_W2, toqk_W, toqk_b, ffn_W1, ffn_W2, mbd, mbd2, vecs):
    B, V, D = x.shape
    R = qkz_W.shape[1] // D
    H = 4
    dh = D // H
    Lqk = qkl_W1.shape[0]
    Lv = vl_W1.shape[0]
    NB = 4 if B % 4 == 0 else 1
    Vf = float(V)
    vid = _vec_index(Lqk, Lv)
    bf16 = jnp.bfloat16

    a_cols = jnp.transpose(a_rows.reshape(R, V, V), (1, 0, 2)).reshape(V, R * V)
    a_cols = a_cols.astype(bf16)
    hid2 = jnp.arange(2 * D, dtype=jnp.int32) // dh
    pool = (hid2[:, None] == jnp.arange(2 * H, dtype=jnp.int32)[None, :])
    pool = pool.astype(bf16)
    poolT = jnp.transpose(pool)

    z3 = z.reshape(B, 1, D)

    consts = dict(
        a_cols=a_cols, vecs=vecs, mbd=mbd,
        pool=pool, poolT=poolT,
        qkz_W=qkz_W, qkz_b=qkz_b,
        qkx_W1a=qkx_W1a.astype(bf16), qkx_W2=qkx_W2.astype(bf16),
        vx_W1a=vx_W1a.astype(bf16), vx_W2=vx_W2.astype(bf16),
        qkl_W1=qkl_W1.astype(bf16), qkl_W2=qkl_W2.astype(bf16),
        vl_zW=vl_zW, vl_zb=vl_zb,
        vl_W1=vl_W1.astype(bf16), vl_W2=vl_W2.astype(bf16),
        toqk_W=toqk_W.astype(bf16), toqk_b=toqk_b,
        ffn_W1=ffn_W1.astype(bf16), ffn_W2=ffn_W2.astype(bf16),
    )
    batched = dict(x=x, z=z3, noise=noise)
    names_b = list(batched.keys())
    names_c = list(consts.keys())
    arrays = {**batched, **consts}
    names = names_b + names_c

    def make_spec(n):
        a = arrays[n]
        nd = a.ndim
        if n in batched:
            return pl.BlockSpec((NB,) + a.shape[1:],
                                lambda b, *_: (b,) + (0,) * (nd - 1))
        return pl.BlockSpec(a.shape, lambda b, *_: (0,) * nd,
                            pipeline_mode=pl.Buffered(1))

    def _ln(t, g, be):
        mu = jnp.mean(t, axis=-1, keepdims=True)
        tc = t - mu
        var = jnp.mean(tc * tc, axis=-1, keepdims=True)
        return tc * jax.lax.rsqrt(var + 1e-5) * g + be

    def _body(h_ref, r_ref, *refs):
        rd = dict(zip(names, refs[:len(names)]))
        o_ref = refs[len(names)]
        base = pl.program_id(0) * NB

        xv = rd["x"][...].reshape(NB * V, D)
        z4 = rd["z"][...].reshape(NB, D)
        nz = rd["noise"][...].reshape(NB * V, 1)
        acols = rd["a_cols"][...]
        vecs_v = rd["vecs"][...]
        mbd_v = rd["mbd"][...]
        pool_v = rd["pool"][...]
        poolT_v = rd["poolT"][...]

        rows = jax.lax.broadcasted_iota(jnp.int32, (NB * V, 1), 0)
        onehot = jnp.zeros((NB * V, 1), jnp.float32)
        kscale_col = jnp.zeros((NB * V, 1), jnp.float32)
        for k in range(NB):
            h_k = h_ref[base + k]
            r_k = r_ref[base + k]
            s_k = jnp.where(jnp.logical_or(r_k == 2, r_k == 3),
                            jnp.float32(0.333), jnp.float32(1.0))
            in_k = jnp.logical_and(rows >= k * V, rows < (k + 1) * V)
            onehot = onehot + (rows == h_k + k * V).astype(jnp.float32)
            kscale_col = kscale_col + s_k * in_k.astype(jnp.float32)

        def vrow(name):
            i = vid[name]
            return vecs_v[i:i + 1, :]

        def bmm(a, w):
            return jnp.dot(a.astype(bf16), w, preferred_element_type=jnp.float32)

        def mlp2(t, w1, b1, w2, b2):
            h = jnp.maximum(bmm(t, w1) + b1, 0.0)
            return bmm(h, w2) + b2

        def rspmm(xcur, zrows):
            outs = []
            for k in range(NB):
                xk = xcur[k * V:(k + 1) * V]
                zk = zrows[k:k + 1]
                xz = jnp.concatenate(
                    [xk * zk[:, r * D:(r + 1) * D] for r in range(R)], axis=0)
                outs.append(jnp.dot(acols, xz.astype(bf16),
                                    preferred_element_type=jnp.float32))
            return jnp.concatenate(outs, axis=0)

        qk_zrows = (jnp.dot(z4, rd["qkz_W"][...],
                            preferred_element_type=jnp.float32)
                    + rd["qkz_b"][...])

        h1 = bmm(xv, rd["qkx_W1a"][...]) + nz * vrow("qkx_w1n") + vrow("qkx_b1")
        h1 = jnp.maximum(h1, 0.0)
        qk_x = bmm(h1, rd["qkx_W2"][...]) + vrow("qkx_b2")

        for l in range(Lqk):
            out = rspmm(qk_x, qk_zrows)
            shortcut = qk_x
            t = out + vrow(f"qkl_alpha_{l}") * qk_x
            t = mlp2(t, rd["qkl_W1"][l], vrow(f"qkl_b1_{l}"),
                     rd["qkl_W2"][l], vrow(f"qkl_b2_{l}"))
            t = _ln(t, vrow(f"qkl_g_{l}"), vrow(f"qkl_be_{l}"))
            qk_x = t + shortcut

        h1 = bmm(xv, rd["vx_W1a"][...]) + onehot * vrow("vx_w1h") + vrow("vx_b1")
        h1 = jnp.maximum(h1, 0.0)
        v_x = bmm(h1, rd["vx_W2"][...]) + vrow("vx_b2")

        for l in range(Lv):
            v_zrows = (jnp.dot(z4, rd["vl_zW"][l],
                               preferred_element_type=jnp.float32)
                       + rd["vl_zb"][l])
            out = rspmm(v_x, v_zrows)
            shortcut = v_x
            t = out + vrow(f"vl_beta_{l}") * v_x
            t = mlp2(t, rd["vl_W1"][l], vrow(f"vl_b1_{l}"),
                     rd["vl_W2"][l], vrow(f"vl_b2_{l}"))
            t = _ln(t, vrow(f"vl_g_{l}"), vrow(f"vl_be_{l}"))
            v_x = t + shortcut

        qk_cat = bmm(qk_x, rd["toqk_W"][...]) + rd["toqk_b"][...]
        s2 = qk_cat * qk_cat
        pooled = bmm(s2, pool_v)
        sq = bmm(pooled, poolT_v)
        qk_n = qk_cat * jax.lax.rsqrt(jnp.maximum(sq, 1e-24))
        qn = qk_n[:, :D]
        kw = qk_n[:, D:] * kscale_col
        poolD = pool_v[:D, :H]
        poolDT = poolT_v[:H, :D]

        attns = []
        for k in range(NB):
            sl = slice(k * V, (k + 1) * V)
            qn_k, kw_k, v_k = qn[sl], kw[sl], v_x[sl]
            kvs = jax.lax.dot_general(kw_k.astype(bf16), v_k.astype(bf16),
                                      (((0,), (0,)), ((), ())),
                                      preferred_element_type=jnp.float32)
            num_mat = bmm(qn_k, (kvs * mbd_v))
            ksum = jnp.sum(kw_k, axis=0, keepdims=True)
            den_p = bmm(qn_k * ksum, poolD)
            den = bmm(den_p, poolDT) + 2.0 * Vf
            vsum = jnp.sum(v_k, axis=0, keepdims=True)
            num = num_mat + vsum + v_k * Vf
            attns.append(num * pl.reciprocal(den, approx=True))
        attn_out = jnp.concatenate(attns, axis=0)

        xa = xv + attn_out
        xa = _ln(xa, vrow("an_g"), vrow("an_b"))
        ffn_out = mlp2(xa, rd["ffn_W1"][...], vrow("ffn_b1"),
                       rd["ffn_W2"][...], vrow("ffn_b2"))
        xo = _ln(xa + ffn_out, vrow("n_g"), vrow("n_b"))
        o_ref[...] = xo.reshape(NB, V, D)

    const_bytes = sum(int(arrays[n].size) * arrays[n].dtype.itemsize
                      for n in names_c)
    step_bytes = (sum(int(arrays[n].size) * arrays[n].dtype.itemsize
                      * NB // B for n in names_b) + NB * V * D * 4)
    vmem_limit = int(min(100 << 20, const_bytes + 4 * step_bytes + (24 << 20)))

    grid_spec = pltpu.PrefetchScalarGridSpec(
        num_scalar_prefetch=2,
        grid=(B // NB,),
        in_specs=[make_spec(n) for n in names],
        out_specs=pl.BlockSpec((NB, V, D), lambda b, *_: (b, 0, 0)),
    )
    return pl.pallas_call(
        _body,
        out_shape=jax.ShapeDtypeStruct((B, V, D), jnp.float32),
        grid_spec=grid_spec,
        compiler_params=pltpu.CompilerParams(
            dimension_semantics=("parallel",),
            vmem_limit_bytes=vmem_limit),
    )(h_index.astype(jnp.int32), r_index.astype(jnp.int32),
      *[arrays[n] for n in names])
